# baseline (speedup 1.0000x reference)
import jax
import jax.numpy as jnp
from jax import lax
from jax.experimental import pallas as pl
from jax.experimental.pallas import tpu as pltpu
from jax.experimental.pallas import tpu_sc as plsc

N = 10000
E = 320000
DIM = 128
H = 8
NPAD = 10240
DUMMY = N
NW = 32
EPW = 10240
EP = NW * EPW
ROWS = EP // 128
RPW = EPW // 128
CH = 8
NCH = RPW // CH
CH2 = 2
NCH2 = RPW // CH2

_mesh = lambda: plsc.VectorSubcoreMesh(core_axis_name="c", subcore_axis_name="s")
_params = lambda: pltpu.CompilerParams(use_tc_tiling_on_sc=False)


def _prolog_body(x_ref, wuu_ref, buu_ref, wvv_ref, tuu_ref, tvv_ref):
    xb = x_ref[...]
    tuu_ref[...] = jnp.dot(xb, wuu_ref[...],
                           preferred_element_type=jnp.float32) + buu_ref[...]
    tvv_ref[...] = jnp.dot(xb, wvv_ref[...], preferred_element_type=jnp.float32)


def _prolog(xp, wuu, buu, wvv):
    return pl.pallas_call(
        _prolog_body,
        grid=(NPAD // 512,),
        in_specs=[
            pl.BlockSpec((512, DIM), lambda i: (i, 0)),
            pl.BlockSpec((DIM, 16), lambda i: (0, 0)),
            pl.BlockSpec((1, 16), lambda i: (0, 0)),
            pl.BlockSpec((DIM, 16), lambda i: (0, 0)),
        ],
        out_specs=[
            pl.BlockSpec((512, 16), lambda i: (i, 0)),
            pl.BlockSpec((512, 16), lambda i: (i, 0)),
        ],
        out_shape=[
            jax.ShapeDtypeStruct((NPAD, 16), jnp.float32),
            jax.ShapeDtypeStruct((NPAD, 16), jnp.float32),
        ],
    )(xp, wuu, buu, wvv)


def _pass1_body(tuu, tvv, srcm, dstm, z16, denoms, exm,
                dsp, sidx, didx, av, bv, semA, semB):
    cid = lax.axis_index("c")
    sid = lax.axis_index("s")
    wid = sid * 2 + cid
    pltpu.sync_copy(z16, dsp.at[pl.ds(sid * 640, 640)])
    plsc.subcore_barrier()

    def _chunk(c, carry):
        r0 = wid * RPW + c * CH
        pltpu.sync_copy(srcm.at[pl.ds(r0, CH)], sidx)
        pltpu.sync_copy(dstm.at[pl.ds(r0, CH)], didx)
        cps = [pltpu.async_copy(tuu.at[sidx.at[j]], av.at[j], semA)
               for j in range(CH)]
        cps += [pltpu.async_copy(tvv.at[didx.at[j]], bv.at[j], semB)
                for j in range(CH)]
        for cp in cps:
            cp.wait()
        for j in range(CH):
            @plsc.parallel_loop(0, 128, 1, unroll=2)
            def _edge(i):
                e2 = av[j, i, :] + bv[j, i, :]
                e2 = jnp.where(e2 >= 0.0, e2, e2 * 0.2)
                av[j, i, :] = jnp.exp(e2)
        outs = [pltpu.async_copy(av.at[j], dsp.at[didx.at[j]], semA, add=True)
                for j in range(CH)]
        outs.append(pltpu.async_copy(av, exm.at[pl.ds(r0, CH)], semB))
        for cp in outs:
            cp.wait()
        return carry
    lax.fori_loop(0, NCH, _chunk, 0)
    plsc.subcore_barrier()
    pltpu.sync_copy(dsp.at[pl.ds(sid * 640, 640)],
                    denoms.at[pl.ds(cid * NPAD + sid * 640, 640)])


def _pass1(tuu, tvv, srcm, dstm, z16):
    return pl.kernel(
        _pass1_body,
        out_type=[
            jax.ShapeDtypeStruct((2 * NPAD, 16), jnp.float32),
            jax.ShapeDtypeStruct((ROWS, 128, 16), jnp.float32),
        ],
        mesh=_mesh(),
        compiler_params=_params(),
        scratch_types=[
            pltpu.VMEM_SHARED((NPAD, 16), jnp.float32),
            pltpu.VMEM((CH, 128), jnp.int32),
            pltpu.VMEM((CH, 128), jnp.int32),
            pltpu.VMEM((CH, 128, 16), jnp.float32),
            pltpu.VMEM((CH, 128, 16), jnp.float32),
            pltpu.SemaphoreType.DMA,
            pltpu.SemaphoreType.DMA,
        ],
    )(tuu, tvv, srcm, dstm, z16)


def _combine_body(denoms, rcp, d0, d1):
    cid = lax.axis_index("c")
    sid = lax.axis_index("s")
    wid = sid * 2 + cid
    r0 = wid * (NPAD // NW)
    pltpu.sync_copy(denoms.at[pl.ds(r0, NPAD // NW)], d0)
    pltpu.sync_copy(denoms.at[pl.ds(NPAD + r0, NPAD // NW)], d1)

    @plsc.parallel_loop(0, NPAD // NW, 1, unroll=2)
    def _row(i):
        d0[i, :] = 1.0 / (d0[i, :] + d1[i, :] + 1e-9)

    pltpu.sync_copy(d0, rcp.at[pl.ds(r0, NPAD // NW)])


def _combine(denoms):
    return pl.kernel(
        _combine_body,
        out_type=jax.ShapeDtypeStruct((NPAD, 16), jnp.float32),
        mesh=_mesh(),
        compiler_params=_params(),
        scratch_types=[
            pltpu.VMEM((NPAD // NW, 16), jnp.float32),
            pltpu.VMEM((NPAD // NW, 16), jnp.float32),
        ],
    )(denoms)


def _pass2_body(rcp, exm, srcm, dstm, xp, z128, aggs,
                asp, sidx, didx, exv, rv, xv, semA, semB):
    cid = lax.axis_index("c")
    sid = lax.axis_index("s")
    wid = sid * 2 + cid
    pltpu.sync_copy(z128, asp.at[pl.ds(sid * 640, 640)])
    plsc.subcore_barrier()

    def _chunk(c, carry):
        r0 = wid * RPW + c * CH2
        pltpu.sync_copy(srcm.at[pl.ds(r0, CH2)], sidx)
        pltpu.sync_copy(dstm.at[pl.ds(r0, CH2)], didx)
        cps = [pltpu.async_copy(exm.at[pl.ds(r0, CH2)], exv, semA)]
        cps += [pltpu.async_copy(rcp.at[didx.at[j]], rv.at[j], semA)
                for j in range(CH2)]
        cps += [pltpu.async_copy(xp.at[sidx.at[j]], xv.at[j], semB)
                for j in range(CH2)]
        for cp in cps:
            cp.wait()
        for j in range(CH2):
            @plsc.parallel_loop(0, 128, 1, unroll=2)
            def _edge(i):
                p2 = exv[j, i, :] * rv[j, i, :]
                for t in range(8):
                    xv[j, i, pl.ds(16 * t, 16)] = xv[j, i, pl.ds(16 * t, 16)] * p2
        outs = [pltpu.async_copy(xv.at[j], asp.at[didx.at[j]], semA, add=True)
                for j in range(CH2)]
        for cp in outs:
            cp.wait()
        return carry
    lax.fori_loop(0, NCH2, _chunk, 0)
    plsc.subcore_barrier()
    pltpu.sync_copy(asp.at[pl.ds(sid * 640, 640)],
                    aggs.at[pl.ds(cid * NPAD + sid * 640, 640)])


def _pass2(rcp, exm, srcm, dstm, xp, z128):
    return pl.kernel(
        _pass2_body,
        out_type=jax.ShapeDtypeStruct((2 * NPAD, DIM), jnp.float32),
        mesh=_mesh(),
        compiler_params=_params(),
        scratch_types=[
            pltpu.VMEM_SHARED((NPAD, DIM), jnp.float32),
            pltpu.VMEM((CH2, 128), jnp.int32),
            pltpu.VMEM((CH2, 128), jnp.int32),
            pltpu.VMEM((CH2, 128, 16), jnp.float32),
            pltpu.VMEM((CH2, 128, 16), jnp.float32),
            pltpu.VMEM((CH2, 128, DIM), jnp.float32),
            pltpu.SemaphoreType.DMA,
            pltpu.SemaphoreType.DMA,
        ],
    )(rcp, exm, srcm, dstm, xp, z128)


def _epilog_body(x_ref, a0_ref, a1_ref, o_ref):
    o_ref[:, :DIM] = x_ref[...]
    o_ref[:, DIM:] = a0_ref[0] + a1_ref[0]


def _epilog(x, aggs3):
    return pl.pallas_call(
        _epilog_body,
        grid=(N // 400,),
        in_specs=[
            pl.BlockSpec((400, DIM), lambda i: (i, 0)),
            pl.BlockSpec((1, 400, DIM), lambda i: (0, i, 0)),
            pl.BlockSpec((1, 400, DIM), lambda i: (1, i, 0)),
        ],
        out_specs=pl.BlockSpec((400, 2 * DIM), lambda i: (i, 0)),
        out_shape=jax.ShapeDtypeStruct((N, 2 * DIM), jnp.float32),
    )(x, aggs3, aggs3)


def kernel(x, edge_index, W_u, b_u, W_v):
    f32 = jnp.float32
    src = edge_index[0]
    dst = edge_index[1]
    xp = jnp.zeros((NPAD, DIM), f32).at[:N].set(x)
    wuu = jnp.concatenate([W_u, W_u], axis=1)
    buu = jnp.concatenate([b_u, b_u]).reshape(1, 16)
    wvv = jnp.concatenate([W_v, W_v], axis=1)
    tuu, tvv = _prolog(xp, wuu, buu, wvv)
    pad = EP - E
    srcm = jnp.concatenate([src, jnp.zeros((pad,), jnp.int32)]).reshape(ROWS, 128)
    dstm = jnp.concatenate([dst, jnp.full((pad,), DUMMY, jnp.int32)]).reshape(ROWS, 128)
    z16 = jnp.zeros((640, 16), f32)
    z128 = jnp.zeros((640, DIM), f32)
    denoms, exm = _pass1(tuu, tvv, srcm, dstm, z16)
    rcp = _combine(denoms)
    aggs = _pass2(rcp, exm, srcm, dstm, xp, z128)
    return _epilog(x, aggs.reshape(2, NPAD, DIM))


# speedup vs baseline: 1.0556x; 1.0556x over previous
import jax
import jax.numpy as jnp
from jax import lax
from jax.experimental import pallas as pl
from jax.experimental.pallas import tpu as pltpu
from jax.experimental.pallas import tpu_sc as plsc

N = 10000
E = 320000
DIM = 128
H = 8
NPAD = 10240
DUMMY = N
NW = 32
EPW = 10240
EP = NW * EPW
ROWS = EP // 128
RPW = EPW // 128
CH = 4
NCH = RPW // CH
CH2 = 2
NCH2 = RPW // CH2

_mesh = lambda: plsc.VectorSubcoreMesh(core_axis_name="c", subcore_axis_name="s")
_params = lambda: pltpu.CompilerParams(use_tc_tiling_on_sc=False)


def _prolog_body(x_ref, wuu_ref, buu_ref, wvv_ref, tuu_ref, tvv_ref):
    xb = x_ref[...]
    tuu_ref[...] = jnp.dot(xb, wuu_ref[...],
                           preferred_element_type=jnp.float32) + buu_ref[...]
    tvv_ref[...] = jnp.dot(xb, wvv_ref[...], preferred_element_type=jnp.float32)


def _prolog(xp, wuu, buu, wvv):
    return pl.pallas_call(
        _prolog_body,
        grid=(NPAD // 512,),
        in_specs=[
            pl.BlockSpec((512, DIM), lambda i: (i, 0)),
            pl.BlockSpec((DIM, 16), lambda i: (0, 0)),
            pl.BlockSpec((1, 16), lambda i: (0, 0)),
            pl.BlockSpec((DIM, 16), lambda i: (0, 0)),
        ],
        out_specs=[
            pl.BlockSpec((512, 16), lambda i: (i, 0)),
            pl.BlockSpec((512, 16), lambda i: (i, 0)),
        ],
        out_shape=[
            jax.ShapeDtypeStruct((NPAD, 16), jnp.float32),
            jax.ShapeDtypeStruct((NPAD, 16), jnp.float32),
        ],
    )(xp, wuu, buu, wvv)


def _pass1_body(tuu, tvv, srcm, dstm, z16, denoms, exm,
                dsp, sidx, didx, av, bv, semA, semB):
    cid = lax.axis_index("c")
    sid = lax.axis_index("s")
    wid = sid * 2 + cid
    pltpu.sync_copy(z16, dsp.at[pl.ds(sid * 640, 640)])
    plsc.subcore_barrier()

    def _chunk(c, carry):
        r0 = wid * RPW + c * CH
        pltpu.sync_copy(srcm.at[pl.ds(r0, CH)], sidx)
        pltpu.sync_copy(dstm.at[pl.ds(r0, CH)], didx)
        cps = [pltpu.async_copy(tuu.at[sidx.at[j]], av.at[j], semA)
               for j in range(CH)]
        cps += [pltpu.async_copy(tvv.at[didx.at[j]], bv.at[j], semB)
                for j in range(CH)]
        for cp in cps:
            cp.wait()
        for j in range(CH):
            @plsc.parallel_loop(0, 128, 1, unroll=2)
            def _edge(i):
                e2 = av[j, i, :] + bv[j, i, :]
                e2 = jnp.where(e2 >= 0.0, e2, e2 * 0.2)
                av[j, i, :] = jnp.exp(e2)
        outs = [pltpu.async_copy(av.at[j], dsp.at[didx.at[j]], semA, add=True)
                for j in range(CH)]
        outs.append(pltpu.async_copy(av, exm.at[pl.ds(r0, CH)], semB))
        for cp in outs:
            cp.wait()
        return carry
    lax.fori_loop(0, NCH, _chunk, 0)
    plsc.subcore_barrier()
    pltpu.sync_copy(dsp.at[pl.ds(sid * 640, 640)],
                    denoms.at[pl.ds(cid * NPAD + sid * 640, 640)])


def _pass1(tuu, tvv, srcm, dstm, z16):
    return pl.kernel(
        _pass1_body,
        out_type=[
            jax.ShapeDtypeStruct((2 * NPAD, 16), jnp.float32),
            jax.ShapeDtypeStruct((ROWS, 128, 16), jnp.float32),
        ],
        mesh=_mesh(),
        compiler_params=_params(),
        scratch_types=[
            pltpu.VMEM_SHARED((NPAD, 16), jnp.float32),
            pltpu.VMEM((CH, 128), jnp.int32),
            pltpu.VMEM((CH, 128), jnp.int32),
            pltpu.VMEM((CH, 128, 16), jnp.float32),
            pltpu.VMEM((CH, 128, 16), jnp.float32),
            pltpu.SemaphoreType.DMA,
            pltpu.SemaphoreType.DMA,
        ],
    )(tuu, tvv, srcm, dstm, z16)


def _combine_body(denoms, rcp, d0, d1):
    cid = lax.axis_index("c")
    sid = lax.axis_index("s")
    wid = sid * 2 + cid
    r0 = wid * (NPAD // NW)
    pltpu.sync_copy(denoms.at[pl.ds(r0, NPAD // NW)], d0)
    pltpu.sync_copy(denoms.at[pl.ds(NPAD + r0, NPAD // NW)], d1)

    @plsc.parallel_loop(0, NPAD // NW, 1, unroll=2)
    def _row(i):
        d0[i, :] = 1.0 / (d0[i, :] + d1[i, :] + 1e-9)

    pltpu.sync_copy(d0, rcp.at[pl.ds(r0, NPAD // NW)])


def _combine(denoms):
    return pl.kernel(
        _combine_body,
        out_type=jax.ShapeDtypeStruct((NPAD, 16), jnp.float32),
        mesh=_mesh(),
        compiler_params=_params(),
        scratch_types=[
            pltpu.VMEM((NPAD // NW, 16), jnp.float32),
            pltpu.VMEM((NPAD // NW, 16), jnp.float32),
        ],
    )(denoms)


def _pass2_body(rcp, exm, srcm, dstm, xp, z128, aggs,
                asp, sidx, didx, exv, rv, xv, semA, semB):
    cid = lax.axis_index("c")
    sid = lax.axis_index("s")
    wid = sid * 2 + cid
    pltpu.sync_copy(z128, asp.at[pl.ds(sid * 640, 640)])
    plsc.subcore_barrier()

    def _chunk(c, carry):
        r0 = wid * RPW + c * CH2
        pltpu.sync_copy(srcm.at[pl.ds(r0, CH2)], sidx)
        pltpu.sync_copy(dstm.at[pl.ds(r0, CH2)], didx)
        cps = [pltpu.async_copy(exm.at[pl.ds(r0, CH2)], exv, semA)]
        cps += [pltpu.async_copy(rcp.at[didx.at[j]], rv.at[j], semA)
                for j in range(CH2)]
        cps += [pltpu.async_copy(xp.at[sidx.at[j]], xv.at[j], semB)
                for j in range(CH2)]
        for cp in cps:
            cp.wait()
        for j in range(CH2):
            @plsc.parallel_loop(0, 128, 1, unroll=2)
            def _edge(i):
                p2 = exv[j, i, :] * rv[j, i, :]
                for t in range(8):
                    xv[j, i, pl.ds(16 * t, 16)] = xv[j, i, pl.ds(16 * t, 16)] * p2
        outs = [pltpu.async_copy(xv.at[j], asp.at[didx.at[j]], semA, add=True)
                for j in range(CH2)]
        for cp in outs:
            cp.wait()
        return carry
    lax.fori_loop(0, NCH2, _chunk, 0)
    plsc.subcore_barrier()
    pltpu.sync_copy(asp.at[pl.ds(sid * 640, 640)],
                    aggs.at[pl.ds(cid * NPAD + sid * 640, 640)])


def _pass2(rcp, exm, srcm, dstm, xp, z128):
    return pl.kernel(
        _pass2_body,
        out_type=jax.ShapeDtypeStruct((2 * NPAD, DIM), jnp.float32),
        mesh=_mesh(),
        compiler_params=_params(),
        scratch_types=[
            pltpu.VMEM_SHARED((NPAD, DIM), jnp.float32),
            pltpu.VMEM((CH2, 128), jnp.int32),
            pltpu.VMEM((CH2, 128), jnp.int32),
            pltpu.VMEM((CH2, 128, 16), jnp.float32),
            pltpu.VMEM((CH2, 128, 16), jnp.float32),
            pltpu.VMEM((CH2, 128, DIM), jnp.float32),
            pltpu.SemaphoreType.DMA,
            pltpu.SemaphoreType.DMA,
        ],
    )(rcp, exm, srcm, dstm, xp, z128)


def _epilog_body(x_ref, a0_ref, a1_ref, o_ref):
    o_ref[:, :DIM] = x_ref[...]
    o_ref[:, DIM:] = a0_ref[0] + a1_ref[0]


def _epilog(x, aggs3):
    return pl.pallas_call(
        _epilog_body,
        grid=(N // 400,),
        in_specs=[
            pl.BlockSpec((400, DIM), lambda i: (i, 0)),
            pl.BlockSpec((1, 400, DIM), lambda i: (0, i, 0)),
            pl.BlockSpec((1, 400, DIM), lambda i: (1, i, 0)),
        ],
        out_specs=pl.BlockSpec((400, 2 * DIM), lambda i: (i, 0)),
        out_shape=jax.ShapeDtypeStruct((N, 2 * DIM), jnp.float32),
    )(x, aggs3, aggs3)


def kernel(x, edge_index, W_u, b_u, W_v):
    f32 = jnp.float32
    src = edge_index[0]
    dst = edge_index[1]
    xp = jnp.zeros((NPAD, DIM), f32).at[:N].set(x)
    wuu = jnp.concatenate([W_u, W_u], axis=1)
    buu = jnp.concatenate([b_u, b_u]).reshape(1, 16)
    wvv = jnp.concatenate([W_v, W_v], axis=1)
    tuu, tvv = _prolog(xp, wuu, buu, wvv)
    pad = EP - E
    srcm = jnp.concatenate([src, jnp.zeros((pad,), jnp.int32)]).reshape(ROWS, 128)
    dstm = jnp.concatenate([dst, jnp.full((pad,), DUMMY, jnp.int32)]).reshape(ROWS, 128)
    z16 = jnp.zeros((640, 16), f32)
    z128 = jnp.zeros((640, DIM), f32)
    denoms, exm = _pass1(tuu, tvv, srcm, dstm, z16)
    rcp = _combine(denoms)
    aggs = _pass2(rcp, exm, srcm, dstm, xp, z128)
    return _epilog(x, aggs.reshape(2, NPAD, DIM))
